# Initial kernel scaffold; baseline (speedup 1.0000x reference)
import jax
import jax.numpy as jnp
from jax.experimental import pallas as pl


def kernel(x):
    raise NotImplementedError("write your pallas kernel here")


# speedup vs baseline: 12.7654x; 12.7654x over previous
import functools

import jax
import jax.numpy as jnp
from jax import lax
from jax.experimental import pallas as pl
from jax.experimental.pallas import tpu as pltpu

B, C, H, W = 2, 5, 721, 1440
K = 50
DX = 25000.0
DY = 25000.0
VORT_THR = 1.4e-4
FILL = -9999.0
NEG = -3.0e38
VALID_CUT = -1.0e38
RM_PAD = 1024


def _pool5(field, op):
    a = field
    for dj in (-2, -1, 1, 2):
        a = op(a, jnp.roll(field, dj, axis=1))
    b = a
    for di in (-2, -1, 1, 2):
        b = op(b, jnp.roll(a, di, axis=0))
    return b


def _mslmin_kernel(msl_ref, mslmin_ref):
    mslmin_ref[0] = _pool5(msl_ref[0, 0], jnp.minimum)


def _w10sq_kernel(u10_ref, v10_ref, w10sq_ref):
    u = u10_ref[0, 0]
    v = v10_ref[0, 0]
    w10sq_ref[0] = _pool5(u * u + v * v, jnp.maximum)


def _peaks_kernel(u850_ref, v850_ref, masked_ref, rowmax_ref):
    u = u850_ref[0, 0]
    v = v850_ref[0, 0]
    du = jnp.concatenate(
        [u[1:2, :] - u[0:1, :],
         (u[2:, :] - u[:-2, :]) / 2.0,
         u[-1:, :] - u[-2:-1, :]], axis=0)
    dv = jnp.concatenate(
        [v[:, 1:2] - v[:, 0:1],
         (v[:, 2:] - v[:, :-2]) / 2.0,
         v[:, -1:] - v[:, -2:-1]], axis=1)
    vort = du / DX + dv / DY
    rm = jnp.maximum(vort, jnp.maximum(jnp.roll(vort, 1, axis=1),
                                       jnp.roll(vort, -1, axis=1)))
    mm = jnp.maximum(rm, jnp.maximum(jnp.roll(rm, 1, axis=0),
                                     jnp.roll(rm, -1, axis=0)))
    peak = (vort >= mm) & (vort > VORT_THR)
    masked = jnp.where(peak, vort, NEG)
    masked_ref[0] = masked
    rowmax = jnp.max(masked, axis=1)
    rowmax_ref[0] = jnp.concatenate(
        [rowmax, jnp.full((RM_PAD - H,), NEG, jnp.float32)]).reshape(1, RM_PAD)


def _select_kernel(masked_ref, rowmax_ref, mslmin_ref, w10sq_ref, frame_ref,
                   work_ref, rm_ref, acc_ref):
    work_ref[...] = masked_ref[0]
    rm_ref[...] = rowmax_ref[0]
    acc_ref[...] = jnp.full((4, 1, RM_PAD), FILL, jnp.float32)

    lane_rm = lax.broadcasted_iota(jnp.int32, (1, RM_PAD), 1)
    lane_w = lax.broadcasted_iota(jnp.int32, (1, W), 1)
    big = jnp.int32(2**30)

    def body(i, _):
        rm = rm_ref[...]
        m = jnp.max(rm)
        r = jnp.min(jnp.where(rm == m, lane_rm, big))
        r = jnp.minimum(r, H - 1)
        row = work_ref[pl.ds(r, 1), :]
        c = jnp.min(jnp.where(row == m, lane_w, big))
        newrow = jnp.where(lane_w == c, NEG, row)
        work_ref[pl.ds(r, 1), :] = newrow
        rm_ref[...] = jnp.where(lane_rm == r, jnp.max(newrow), rm)

        valid = m > VALID_CUT
        latv = 90.0 - 0.25 * r.astype(jnp.float32)
        lonv = 0.25 * c.astype(jnp.float32)
        mrow = mslmin_ref[0, pl.ds(r, 1), :]
        mslv = jnp.sum(jnp.where(lane_w == c, mrow, 0.0))
        wrow = w10sq_ref[0, pl.ds(r, 1), :]
        w10v = jnp.sqrt(jnp.sum(jnp.where(lane_w == c, wrow, 0.0)))
        sel = lane_rm == i
        for j, val in enumerate((latv, lonv, mslv, w10v)):
            acc_ref[j] = jnp.where(
                sel, jnp.where(valid, val, FILL), acc_ref[j])
        return 0

    lax.fori_loop(0, K, body, 0)
    frame_ref[0] = acc_ref[...].reshape(4, RM_PAD)


def _ch_spec(ch):
    return pl.BlockSpec((1, 1, H, W), lambda b: (b, ch, 0, 0))


@jax.jit
def kernel(x):
    f32 = jnp.float32
    mslmin = pl.pallas_call(
        _mslmin_kernel,
        grid=(B,),
        in_specs=[_ch_spec(2)],
        out_specs=pl.BlockSpec((1, H, W), lambda b: (b, 0, 0)),
        out_shape=jax.ShapeDtypeStruct((B, H, W), f32),
    )(x)
    w10sq = pl.pallas_call(
        _w10sq_kernel,
        grid=(B,),
        in_specs=[_ch_spec(0), _ch_spec(1)],
        out_specs=pl.BlockSpec((1, H, W), lambda b: (b, 0, 0)),
        out_shape=jax.ShapeDtypeStruct((B, H, W), f32),
    )(x, x)

    masked, rowmax = pl.pallas_call(
        _peaks_kernel,
        grid=(B,),
        in_specs=[_ch_spec(3), _ch_spec(4)],
        out_specs=[pl.BlockSpec((1, H, W), lambda b: (b, 0, 0)),
                   pl.BlockSpec((1, 1, RM_PAD), lambda b: (b, 0, 0))],
        out_shape=[jax.ShapeDtypeStruct((B, H, W), f32),
                   jax.ShapeDtypeStruct((B, 1, RM_PAD), f32)],
    )(x, x)

    frame4 = pl.pallas_call(
        _select_kernel,
        grid=(B,),
        in_specs=[pl.BlockSpec((1, H, W), lambda b: (b, 0, 0)),
                  pl.BlockSpec((1, 1, RM_PAD), lambda b: (b, 0, 0)),
                  pl.BlockSpec((1, H, W), lambda b: (b, 0, 0)),
                  pl.BlockSpec((1, H, W), lambda b: (b, 0, 0))],
        out_specs=pl.BlockSpec((1, 4, RM_PAD), lambda b: (b, 0, 0)),
        out_shape=jax.ShapeDtypeStruct((B, 4, RM_PAD), f32),
        scratch_shapes=[pltpu.VMEM((H, W), f32),
                        pltpu.VMEM((1, RM_PAD), f32),
                        pltpu.VMEM((4, 1, RM_PAD), f32)],
    )(masked, rowmax, mslmin, w10sq)

    return frame4[:, :, :K].transpose(0, 2, 1)
